# baseline (speedup 1.0000x reference)
import functools
import jax
import jax.numpy as jnp
from jax import lax
from jax.experimental import pallas as pl
from jax.experimental.pallas import tpu as pltpu
from jax.experimental.pallas import tpu_sc as plsc

N = 10000
E = 320000
D_IN = 128
D_HID = 64
D_OUT = 128

NC = 2
NS = 16
NW = NC * NS
BATCH = 256
K = 2
N_PAD = 10240
E_PAD = 327680
ROWS_PER_TILE = E_PAD // (NW * BATCH)
PAIRS = ROWS_PER_TILE // (2 * K)
P0 = 12
P1 = 2 * PAIRS - P0
PMAX = max(P0, P1)
ROWS_NP = N_PAD // NS
DEG_ROWS = ROWS_PER_TILE

_mesh = plsc.VectorSubcoreMesh(
    core_axis_name="c", subcore_axis_name="s", num_cores=NC, num_subcores=NS)


def _sc_seg_body(z_hbm, src_hbm, dst_hbm, zeros64_hbm, s_out,
                 srcv, dstv, rowsv, acc, sem):
  cid = lax.axis_index("c")
  sid = lax.axis_index("s")
  n_pairs = jnp.where(cid == 0, P0, P1)
  base = jnp.where(cid == 0, sid * (2 * K * P0),
                   NS * 2 * K * P0 + sid * (2 * K * P1))
  rows_n = 2 * K * n_pairs

  pltpu.sync_copy(zeros64_hbm.at[pl.ds(sid * ROWS_NP, ROWS_NP)],
                  acc.at[pl.ds(sid * ROWS_NP, ROWS_NP)])
  pltpu.sync_copy(src_hbm.at[pl.ds(base, 2 * K * PMAX)], srcv)
  pltpu.sync_copy(dst_hbm.at[pl.ds(base, 2 * K * PMAX)], dstv)
  plsc.subcore_barrier()

  def group(g, carry):
    for b in range(K):
      r = lax.rem(g * K + b, rows_n)
      pltpu.async_copy(z_hbm.at[srcv.at[r]], rowsv.at[b], sem)
    for b in range(K):
      r = lax.rem(g * K + b, rows_n)
      pltpu.make_async_copy(z_hbm.at[srcv.at[r]], rowsv.at[b], sem).wait()
    for b in range(K):
      pltpu.sync_copy(rowsv.at[b], acc.at[dstv.at[g * K + b]], add=True)
    return carry

  lax.fori_loop(0, 2 * n_pairs, group, 0)
  plsc.subcore_barrier()

  pltpu.sync_copy(acc.at[pl.ds(sid * ROWS_NP, ROWS_NP)],
                  s_out.at[pl.ds(cid * N_PAD + sid * ROWS_NP, ROWS_NP)])


_sc_pass = pl.kernel(
    _sc_seg_body,
    out_type=[jax.ShapeDtypeStruct((NC * N_PAD, D_HID), jnp.float32)],
    mesh=_mesh,
    scratch_types=[
        pltpu.VMEM((2 * K * PMAX, BATCH), jnp.int32),
        pltpu.VMEM((2 * K * PMAX, BATCH), jnp.int32),
        pltpu.VMEM((K, BATCH, D_HID), jnp.float32),
        pltpu.VMEM_SHARED((N_PAD, D_HID), jnp.float32),
        pltpu.SemaphoreType.DMA,
    ],
    compiler_params=pltpu.CompilerParams(use_tc_tiling_on_sc=False))


def _sc_deg_body(dst_hbm, zeros16_hbm, ones16_hbm, deg_out,
                 dstv, onesv, dacc):
  cid = lax.axis_index("c")
  sid = lax.axis_index("s")
  wid = cid * NS + sid

  pltpu.sync_copy(zeros16_hbm.at[pl.ds(sid * ROWS_NP, ROWS_NP)],
                  dacc.at[pl.ds(sid * ROWS_NP, ROWS_NP)])
  pltpu.sync_copy(dst_hbm.at[pl.ds(wid * DEG_ROWS, DEG_ROWS)], dstv)
  pltpu.sync_copy(ones16_hbm, onesv)
  plsc.subcore_barrier()

  def group(r, carry):
    pltpu.sync_copy(onesv, dacc.at[dstv.at[r]], add=True)
    return carry

  lax.fori_loop(0, DEG_ROWS, group, 0)
  plsc.subcore_barrier()

  pltpu.sync_copy(dacc.at[pl.ds(sid * ROWS_NP, ROWS_NP)],
                  deg_out.at[pl.ds(cid * N_PAD + sid * ROWS_NP, ROWS_NP)])


_sc_deg = pl.kernel(
    _sc_deg_body,
    out_type=[jax.ShapeDtypeStruct((NC * N_PAD, 16), jnp.float32)],
    mesh=_mesh,
    scratch_types=[
        pltpu.VMEM((DEG_ROWS, BATCH), jnp.int32),
        pltpu.VMEM((BATCH, 16), jnp.float32),
        pltpu.VMEM_SHARED((N_PAD, 16), jnp.float32),
    ],
    compiler_params=pltpu.CompilerParams(use_tc_tiling_on_sc=False))


def _inv_deg(deg_ref):
  d = deg_ref[0:N, 0:1] + deg_ref[N_PAD:N_PAD + N, 0:1]
  return 1.0 / jnp.maximum(d, 1.0)


def _bn_cols(h, g, b, eps=1e-5):
  mu = jnp.mean(h, axis=0, keepdims=True)
  var = jnp.mean((h - mu) * (h - mu), axis=0, keepdims=True)
  return (h - mu) * lax.rsqrt(var + eps) * g + b


def _tc_a_body(x_ref, ws_ref, wn_ref, o1_ref, o2_ref):
  x = x_ref[...]
  o1_ref[...] = jnp.dot(x, ws_ref[...], preferred_element_type=jnp.float32)
  o2_ref[...] = jnp.dot(x, wn_ref[...], preferred_element_type=jnp.float32)


def _tc_mid_body(relu, hlin_ref, s_ref, deg_ref, b_ref, g_ref, bt_ref,
                 ws_ref, wn_ref, o1_ref, o2_ref):
  inv = _inv_deg(deg_ref)
  agg = (s_ref[0:N, :] + s_ref[N_PAD:N_PAD + N, :]) * inv
  h = hlin_ref[...] + agg + b_ref[...][None, :]
  h = _bn_cols(h, g_ref[...][None, :], bt_ref[...][None, :])
  if relu:
    h = jnp.maximum(h, 0.0)
  o1_ref[...] = jnp.dot(h, ws_ref[...], preferred_element_type=jnp.float32)
  o2_ref[...] = jnp.dot(h, wn_ref[...], preferred_element_type=jnp.float32)


def _tc_d_body(hlin_ref, s_ref, deg_ref, b_ref, g_ref, bt_ref, ws_ref,
               o1_ref, o2_ref):
  inv = _inv_deg(deg_ref)
  agg = (s_ref[0:N, :] + s_ref[N_PAD:N_PAD + N, :]) * inv
  h = hlin_ref[...] + agg + b_ref[...][None, :]
  h = _bn_cols(h, g_ref[...][None, :], bt_ref[...][None, :])
  h = jnp.maximum(h, 0.0)
  o1_ref[...] = jnp.dot(h, ws_ref[...], preferred_element_type=jnp.float32)
  o2_ref[...] = h


def _tc_e_body(hlin_ref, s_ref, deg_ref, wn_ref, bo_ref, o_ref):
  inv = _inv_deg(deg_ref)
  agg = (s_ref[0:N, :] + s_ref[N_PAD:N_PAD + N, :]) * inv
  o_ref[...] = (hlin_ref[...]
                + jnp.dot(agg, wn_ref[...], preferred_element_type=jnp.float32)
                + bo_ref[...][None, :])


def _sds(shape):
  return jax.ShapeDtypeStruct(shape, jnp.float32)


def kernel(features, edge_index, Ws1, Wn1, b1, g1, bt1, Ws2, Wn2, b2, g2, bt2,
           Ws3, Wn3, b3, g3, bt3, Wso, Wno, bo):
  src = edge_index[0]
  dst = edge_index[1]
  pad = E_PAD - E
  src2d = jnp.concatenate([src, jnp.zeros((pad,), jnp.int32)]).reshape(-1, BATCH)
  dst2d = jnp.concatenate([dst, jnp.full((pad,), N, jnp.int32)]).reshape(-1, BATCH)
  zeros64 = jnp.zeros((N_PAD, D_HID), jnp.float32)
  zeros16 = jnp.zeros((N_PAD, 16), jnp.float32)
  ones16 = jnp.ones((BATCH, 16), jnp.float32)

  tc_a = pl.pallas_call(_tc_a_body, out_shape=[_sds((N, D_HID))] * 2)
  tc_mid_relu = pl.pallas_call(functools.partial(_tc_mid_body, True),
                               out_shape=[_sds((N, D_HID))] * 2)
  tc_mid_norelu = pl.pallas_call(functools.partial(_tc_mid_body, False),
                                 out_shape=[_sds((N, D_HID))] * 2)
  tc_d = pl.pallas_call(_tc_d_body,
                        out_shape=[_sds((N, D_OUT)), _sds((N, D_HID))])
  tc_e = pl.pallas_call(_tc_e_body, out_shape=_sds((N, D_OUT)))

  xWs1, z1 = tc_a(features, Ws1, Wn1)
  (deg,) = _sc_deg(dst2d, zeros16, ones16)
  (s1,) = _sc_pass(z1, src2d, dst2d, zeros64)
  hWs2, z2 = tc_mid_norelu(xWs1, s1, deg, b1, g1, bt1, Ws2, Wn2)
  (s2,) = _sc_pass(z2, src2d, dst2d, zeros64)
  hWs3, z3 = tc_mid_relu(hWs2, s2, deg, b2, g2, bt2, Ws3, Wn3)
  (s3,) = _sc_pass(z3, src2d, dst2d, zeros64)
  h3Wso, h3 = tc_d(hWs3, s3, deg, b3, g3, bt3, Wso)
  (s4,) = _sc_pass(h3, src2d, dst2d, zeros64)
  return tc_e(h3Wso, s4, deg, Wno, bo)


# speedup vs baseline: 1.1582x; 1.1582x over previous
import functools
import jax
import jax.numpy as jnp
from jax import lax
from jax.experimental import pallas as pl
from jax.experimental.pallas import tpu as pltpu
from jax.experimental.pallas import tpu_sc as plsc

N = 10000
E = 320000
D_IN = 128
D_HID = 64
D_OUT = 128

NC = 2
NS = 16
NW = NC * NS
BATCH = 256
K = 2
N_PAD = 10240
E_PAD = 327680
ROWS_PER_TILE = E_PAD // (NW * BATCH)
PAIRS = ROWS_PER_TILE // (2 * K)
P0 = 17
P1 = 2 * PAIRS - P0
PMAX = max(P0, P1)
ROWS_NP = N_PAD // NS
DEG_ROWS = ROWS_PER_TILE

_mesh = plsc.VectorSubcoreMesh(
    core_axis_name="c", subcore_axis_name="s", num_cores=NC, num_subcores=NS)


def _sc_seg_body(z_hbm, src_hbm, dst_hbm, zeros64_hbm, s_out,
                 srcv, dstv, rowsv, acc, sem):
  cid = lax.axis_index("c")
  sid = lax.axis_index("s")
  n_pairs = jnp.where(cid == 0, P0, P1)
  base = jnp.where(cid == 0, sid * (2 * K * P0),
                   NS * 2 * K * P0 + sid * (2 * K * P1))
  rows_n = 2 * K * n_pairs

  pltpu.sync_copy(zeros64_hbm.at[pl.ds(sid * ROWS_NP, ROWS_NP)],
                  acc.at[pl.ds(sid * ROWS_NP, ROWS_NP)])
  pltpu.sync_copy(src_hbm.at[pl.ds(base, 2 * K * PMAX)], srcv)
  pltpu.sync_copy(dst_hbm.at[pl.ds(base, 2 * K * PMAX)], dstv)
  plsc.subcore_barrier()

  def group(g, carry):
    for b in range(K):
      r = lax.rem(g * K + b, rows_n)
      pltpu.async_copy(z_hbm.at[srcv.at[r]], rowsv.at[b], sem)
    for b in range(K):
      r = lax.rem(g * K + b, rows_n)
      pltpu.make_async_copy(z_hbm.at[srcv.at[r]], rowsv.at[b], sem).wait()
    for b in range(K):
      pltpu.sync_copy(rowsv.at[b], acc.at[dstv.at[g * K + b]], add=True)
    return carry

  lax.fori_loop(0, 2 * n_pairs, group, 0)
  plsc.subcore_barrier()

  pltpu.sync_copy(acc.at[pl.ds(sid * ROWS_NP, ROWS_NP)],
                  s_out.at[pl.ds(cid * N_PAD + sid * ROWS_NP, ROWS_NP)])


_sc_pass = pl.kernel(
    _sc_seg_body,
    out_type=[jax.ShapeDtypeStruct((NC * N_PAD, D_HID), jnp.float32)],
    mesh=_mesh,
    scratch_types=[
        pltpu.VMEM((2 * K * PMAX, BATCH), jnp.int32),
        pltpu.VMEM((2 * K * PMAX, BATCH), jnp.int32),
        pltpu.VMEM((K, BATCH, D_HID), jnp.float32),
        pltpu.VMEM_SHARED((N_PAD, D_HID), jnp.float32),
        pltpu.SemaphoreType.DMA,
    ],
    compiler_params=pltpu.CompilerParams(use_tc_tiling_on_sc=False))


def _sc_deg_body(dst_hbm, zeros16_hbm, ones16_hbm, deg_out,
                 dstv, onesv, dacc):
  cid = lax.axis_index("c")
  sid = lax.axis_index("s")
  wid = cid * NS + sid

  pltpu.sync_copy(zeros16_hbm.at[pl.ds(sid * ROWS_NP, ROWS_NP)],
                  dacc.at[pl.ds(sid * ROWS_NP, ROWS_NP)])
  pltpu.sync_copy(dst_hbm.at[pl.ds(wid * DEG_ROWS, DEG_ROWS)], dstv)
  pltpu.sync_copy(ones16_hbm, onesv)
  plsc.subcore_barrier()

  def group(r, carry):
    pltpu.sync_copy(onesv, dacc.at[dstv.at[r]], add=True)
    return carry

  lax.fori_loop(0, DEG_ROWS, group, 0)
  plsc.subcore_barrier()

  pltpu.sync_copy(dacc.at[pl.ds(sid * ROWS_NP, ROWS_NP)],
                  deg_out.at[pl.ds(cid * N_PAD + sid * ROWS_NP, ROWS_NP)])


_sc_deg = pl.kernel(
    _sc_deg_body,
    out_type=[jax.ShapeDtypeStruct((NC * N_PAD, 16), jnp.float32)],
    mesh=_mesh,
    scratch_types=[
        pltpu.VMEM((DEG_ROWS, BATCH), jnp.int32),
        pltpu.VMEM((BATCH, 16), jnp.float32),
        pltpu.VMEM_SHARED((N_PAD, 16), jnp.float32),
    ],
    compiler_params=pltpu.CompilerParams(use_tc_tiling_on_sc=False))


def _inv_deg(deg_ref):
  d = deg_ref[0:N, 0:1] + deg_ref[N_PAD:N_PAD + N, 0:1]
  return 1.0 / jnp.maximum(d, 1.0)


def _bn_cols(h, g, b, eps=1e-5):
  mu = jnp.mean(h, axis=0, keepdims=True)
  var = jnp.mean((h - mu) * (h - mu), axis=0, keepdims=True)
  return (h - mu) * lax.rsqrt(var + eps) * g + b


def _tc_a_body(x_ref, ws_ref, wn_ref, o1_ref, o2_ref):
  x = x_ref[...]
  o1_ref[...] = jnp.dot(x, ws_ref[...], preferred_element_type=jnp.float32)
  o2_ref[...] = jnp.dot(x, wn_ref[...], preferred_element_type=jnp.float32)


def _tc_mid_body(relu, hlin_ref, s_ref, deg_ref, b_ref, g_ref, bt_ref,
                 ws_ref, wn_ref, o1_ref, o2_ref):
  inv = _inv_deg(deg_ref)
  agg = (s_ref[0:N, :] + s_ref[N_PAD:N_PAD + N, :]) * inv
  h = hlin_ref[...] + agg + b_ref[...][None, :]
  h = _bn_cols(h, g_ref[...][None, :], bt_ref[...][None, :])
  if relu:
    h = jnp.maximum(h, 0.0)
  o1_ref[...] = jnp.dot(h, ws_ref[...], preferred_element_type=jnp.float32)
  o2_ref[...] = jnp.dot(h, wn_ref[...], preferred_element_type=jnp.float32)


def _tc_d_body(hlin_ref, s_ref, deg_ref, b_ref, g_ref, bt_ref, ws_ref,
               o1_ref, o2_ref):
  inv = _inv_deg(deg_ref)
  agg = (s_ref[0:N, :] + s_ref[N_PAD:N_PAD + N, :]) * inv
  h = hlin_ref[...] + agg + b_ref[...][None, :]
  h = _bn_cols(h, g_ref[...][None, :], bt_ref[...][None, :])
  h = jnp.maximum(h, 0.0)
  o1_ref[...] = jnp.dot(h, ws_ref[...], preferred_element_type=jnp.float32)
  o2_ref[...] = h


def _tc_e_body(hlin_ref, s_ref, deg_ref, wn_ref, bo_ref, o_ref):
  inv = _inv_deg(deg_ref)
  agg = (s_ref[0:N, :] + s_ref[N_PAD:N_PAD + N, :]) * inv
  o_ref[...] = (hlin_ref[...]
                + jnp.dot(agg, wn_ref[...], preferred_element_type=jnp.float32)
                + bo_ref[...][None, :])


def _sds(shape):
  return jax.ShapeDtypeStruct(shape, jnp.float32)


def kernel(features, edge_index, Ws1, Wn1, b1, g1, bt1, Ws2, Wn2, b2, g2, bt2,
           Ws3, Wn3, b3, g3, bt3, Wso, Wno, bo):
  src = edge_index[0]
  dst = edge_index[1]
  pad = E_PAD - E
  src2d = jnp.concatenate([src, jnp.zeros((pad,), jnp.int32)]).reshape(-1, BATCH)
  dst2d = jnp.concatenate([dst, jnp.full((pad,), N, jnp.int32)]).reshape(-1, BATCH)
  zeros64 = jnp.zeros((N_PAD, D_HID), jnp.float32)
  zeros16 = jnp.zeros((N_PAD, 16), jnp.float32)
  ones16 = jnp.ones((BATCH, 16), jnp.float32)

  tc_a = pl.pallas_call(_tc_a_body, out_shape=[_sds((N, D_HID))] * 2)
  tc_mid_relu = pl.pallas_call(functools.partial(_tc_mid_body, True),
                               out_shape=[_sds((N, D_HID))] * 2)
  tc_mid_norelu = pl.pallas_call(functools.partial(_tc_mid_body, False),
                                 out_shape=[_sds((N, D_HID))] * 2)
  tc_d = pl.pallas_call(_tc_d_body,
                        out_shape=[_sds((N, D_OUT)), _sds((N, D_HID))])
  tc_e = pl.pallas_call(_tc_e_body, out_shape=_sds((N, D_OUT)))

  xWs1, z1 = tc_a(features, Ws1, Wn1)
  (deg,) = _sc_deg(dst2d, zeros16, ones16)
  (s1,) = _sc_pass(z1, src2d, dst2d, zeros64)
  hWs2, z2 = tc_mid_norelu(xWs1, s1, deg, b1, g1, bt1, Ws2, Wn2)
  (s2,) = _sc_pass(z2, src2d, dst2d, zeros64)
  hWs3, z3 = tc_mid_relu(hWs2, s2, deg, b2, g2, bt2, Ws3, Wn3)
  (s3,) = _sc_pass(z3, src2d, dst2d, zeros64)
  h3Wso, h3 = tc_d(hWs3, s3, deg, b3, g3, bt3, Wso)
  (s4,) = _sc_pass(h3, src2d, dst2d, zeros64)
  return tc_e(h3Wso, s4, deg, Wno, bo)


# speedup vs baseline: 1.3035x; 1.1255x over previous
import functools
import jax
import jax.numpy as jnp
from jax import lax
from jax.experimental import pallas as pl
from jax.experimental.pallas import tpu as pltpu
from jax.experimental.pallas import tpu_sc as plsc

N = 10000
E = 320000
D_IN = 128
D_HID = 64
D_OUT = 128

NC = 2
NS = 16
NW = NC * NS
BATCH = 256
K = 2
N_PAD = 10240
E_PAD = 327680
ROWS_PER_TILE = E_PAD // (NW * BATCH)
PAIRS = ROWS_PER_TILE // (2 * K)
P0 = 19
P1 = 2 * PAIRS - P0
PMAX = max(P0, P1)
ROWS_NP = N_PAD // NS
DEG_ROWS = ROWS_PER_TILE

_mesh = plsc.VectorSubcoreMesh(
    core_axis_name="c", subcore_axis_name="s", num_cores=NC, num_subcores=NS)


def _sc_seg_body(z_hbm, src_hbm, dst_hbm, zeros64_hbm, s_out,
                 srcv, dstv, rowsv, acc, sem):
  cid = lax.axis_index("c")
  sid = lax.axis_index("s")
  n_pairs = jnp.where(cid == 0, P0, P1)
  base = jnp.where(cid == 0, sid * (2 * K * P0),
                   NS * 2 * K * P0 + sid * (2 * K * P1))
  rows_n = 2 * K * n_pairs

  pltpu.sync_copy(zeros64_hbm.at[pl.ds(sid * ROWS_NP, ROWS_NP)],
                  acc.at[pl.ds(sid * ROWS_NP, ROWS_NP)])
  pltpu.sync_copy(src_hbm.at[pl.ds(base, 2 * K * PMAX)], srcv)
  pltpu.sync_copy(dst_hbm.at[pl.ds(base, 2 * K * PMAX)], dstv)
  plsc.subcore_barrier()

  def group(g, carry):
    for b in range(K):
      r = lax.rem(g * K + b, rows_n)
      pltpu.async_copy(z_hbm.at[srcv.at[r]], rowsv.at[b], sem)
    for b in range(K):
      r = lax.rem(g * K + b, rows_n)
      pltpu.make_async_copy(z_hbm.at[srcv.at[r]], rowsv.at[b], sem).wait()
    for b in range(K):
      pltpu.sync_copy(rowsv.at[b], acc.at[dstv.at[g * K + b]], add=True)
    return carry

  lax.fori_loop(0, 2 * n_pairs, group, 0)
  plsc.subcore_barrier()

  pltpu.sync_copy(acc.at[pl.ds(sid * ROWS_NP, ROWS_NP)],
                  s_out.at[pl.ds(cid * N_PAD + sid * ROWS_NP, ROWS_NP)])


_sc_pass = pl.kernel(
    _sc_seg_body,
    out_type=[jax.ShapeDtypeStruct((NC * N_PAD, D_HID), jnp.float32)],
    mesh=_mesh,
    scratch_types=[
        pltpu.VMEM((2 * K * PMAX, BATCH), jnp.int32),
        pltpu.VMEM((2 * K * PMAX, BATCH), jnp.int32),
        pltpu.VMEM((K, BATCH, D_HID), jnp.float32),
        pltpu.VMEM_SHARED((N_PAD, D_HID), jnp.float32),
        pltpu.SemaphoreType.DMA,
    ],
    compiler_params=pltpu.CompilerParams(use_tc_tiling_on_sc=False))


def _sc_deg_body(dst_hbm, zeros16_hbm, ones16_hbm, deg_out,
                 dstv, onesv, dacc):
  cid = lax.axis_index("c")
  sid = lax.axis_index("s")
  wid = cid * NS + sid

  pltpu.sync_copy(zeros16_hbm.at[pl.ds(sid * ROWS_NP, ROWS_NP)],
                  dacc.at[pl.ds(sid * ROWS_NP, ROWS_NP)])
  pltpu.sync_copy(dst_hbm.at[pl.ds(wid * DEG_ROWS, DEG_ROWS)], dstv)
  pltpu.sync_copy(ones16_hbm, onesv)
  plsc.subcore_barrier()

  def group(r, carry):
    pltpu.sync_copy(onesv, dacc.at[dstv.at[r]], add=True)
    return carry

  lax.fori_loop(0, DEG_ROWS, group, 0)
  plsc.subcore_barrier()

  pltpu.sync_copy(dacc.at[pl.ds(sid * ROWS_NP, ROWS_NP)],
                  deg_out.at[pl.ds(cid * N_PAD + sid * ROWS_NP, ROWS_NP)])


_sc_deg = pl.kernel(
    _sc_deg_body,
    out_type=[jax.ShapeDtypeStruct((NC * N_PAD, 16), jnp.float32)],
    mesh=_mesh,
    scratch_types=[
        pltpu.VMEM((DEG_ROWS, BATCH), jnp.int32),
        pltpu.VMEM((BATCH, 16), jnp.float32),
        pltpu.VMEM_SHARED((N_PAD, 16), jnp.float32),
    ],
    compiler_params=pltpu.CompilerParams(use_tc_tiling_on_sc=False))


def _inv_deg(deg_ref):
  d = deg_ref[0:N, 0:1] + deg_ref[N_PAD:N_PAD + N, 0:1]
  return 1.0 / jnp.maximum(d, 1.0)


def _bn_cols(h, g, b, eps=1e-5):
  mu = jnp.mean(h, axis=0, keepdims=True)
  var = jnp.mean((h - mu) * (h - mu), axis=0, keepdims=True)
  return (h - mu) * lax.rsqrt(var + eps) * g + b


def _tc_a_body(x_ref, ws_ref, wn_ref, o1_ref, o2_ref):
  x = x_ref[...]
  o1_ref[...] = jnp.dot(x, ws_ref[...], preferred_element_type=jnp.float32)
  o2_ref[...] = jnp.dot(x, wn_ref[...], preferred_element_type=jnp.float32)


def _tc_mid_body(relu, hlin_ref, s_ref, deg_ref, b_ref, g_ref, bt_ref,
                 ws_ref, wn_ref, o1_ref, o2_ref):
  inv = _inv_deg(deg_ref)
  agg = (s_ref[0:N, :] + s_ref[N_PAD:N_PAD + N, :]) * inv
  h = hlin_ref[...] + agg + b_ref[...][None, :]
  h = _bn_cols(h, g_ref[...][None, :], bt_ref[...][None, :])
  if relu:
    h = jnp.maximum(h, 0.0)
  o1_ref[...] = jnp.dot(h, ws_ref[...], preferred_element_type=jnp.float32)
  o2_ref[...] = jnp.dot(h, wn_ref[...], preferred_element_type=jnp.float32)


def _tc_d_body(hlin_ref, s_ref, deg_ref, b_ref, g_ref, bt_ref, ws_ref,
               o1_ref, o2_ref):
  inv = _inv_deg(deg_ref)
  agg = (s_ref[0:N, :] + s_ref[N_PAD:N_PAD + N, :]) * inv
  h = hlin_ref[...] + agg + b_ref[...][None, :]
  h = _bn_cols(h, g_ref[...][None, :], bt_ref[...][None, :])
  h = jnp.maximum(h, 0.0)
  o1_ref[...] = jnp.dot(h, ws_ref[...], preferred_element_type=jnp.float32)
  o2_ref[...] = h


def _tc_e_body(hlin_ref, s_ref, deg_ref, wn_ref, bo_ref, o_ref):
  inv = _inv_deg(deg_ref)
  agg = (s_ref[0:N, :] + s_ref[N_PAD:N_PAD + N, :]) * inv
  o_ref[...] = (hlin_ref[...]
                + jnp.dot(agg, wn_ref[...], preferred_element_type=jnp.float32)
                + bo_ref[...][None, :])


def _sds(shape):
  return jax.ShapeDtypeStruct(shape, jnp.float32)


def kernel(features, edge_index, Ws1, Wn1, b1, g1, bt1, Ws2, Wn2, b2, g2, bt2,
           Ws3, Wn3, b3, g3, bt3, Wso, Wno, bo):
  src = edge_index[0]
  dst = edge_index[1]
  pad = E_PAD - E
  src2d = jnp.concatenate([src, jnp.zeros((pad,), jnp.int32)]).reshape(-1, BATCH)
  dst2d = jnp.concatenate([dst, jnp.full((pad,), N, jnp.int32)]).reshape(-1, BATCH)
  zeros64 = jnp.zeros((N_PAD, D_HID), jnp.float32)
  zeros16 = jnp.zeros((N_PAD, 16), jnp.float32)
  ones16 = jnp.ones((BATCH, 16), jnp.float32)

  tc_a = pl.pallas_call(_tc_a_body, out_shape=[_sds((N, D_HID))] * 2)
  tc_mid_relu = pl.pallas_call(functools.partial(_tc_mid_body, True),
                               out_shape=[_sds((N, D_HID))] * 2)
  tc_mid_norelu = pl.pallas_call(functools.partial(_tc_mid_body, False),
                                 out_shape=[_sds((N, D_HID))] * 2)
  tc_d = pl.pallas_call(_tc_d_body,
                        out_shape=[_sds((N, D_OUT)), _sds((N, D_HID))])
  tc_e = pl.pallas_call(_tc_e_body, out_shape=_sds((N, D_OUT)))

  xWs1, z1 = tc_a(features, Ws1, Wn1)
  (deg,) = _sc_deg(dst2d, zeros16, ones16)
  (s1,) = _sc_pass(z1, src2d, dst2d, zeros64)
  hWs2, z2 = tc_mid_norelu(xWs1, s1, deg, b1, g1, bt1, Ws2, Wn2)
  (s2,) = _sc_pass(z2, src2d, dst2d, zeros64)
  hWs3, z3 = tc_mid_relu(hWs2, s2, deg, b2, g2, bt2, Ws3, Wn3)
  (s3,) = _sc_pass(z3, src2d, dst2d, zeros64)
  h3Wso, h3 = tc_d(hWs3, s3, deg, b3, g3, bt3, Wso)
  (s4,) = _sc_pass(h3, src2d, dst2d, zeros64)
  return tc_e(h3Wso, s4, deg, Wno, bo)
